# baseline (speedup 1.0000x reference)
import jax
import jax.numpy as jnp
from jax import lax
from jax.experimental import pallas as pl
from jax.experimental.pallas import tpu as pltpu

_K = 1024
_D = 64
_T = 1024


def _vq_body(z_ref, w_ref, q_ref, zq_ref):
    z = z_ref[0]
    w = w_ref[...]
    s = lax.dot_general(z, w, (((0,), (1,)), ((), ())),
                        preferred_element_type=jnp.float32)
    z2 = jnp.sum(z * z, axis=0)
    w2 = jnp.sum(w * w, axis=1)
    dist = (z2[:, None] - 2.0 * s) + w2[None, :]
    mind = jnp.min(dist, axis=1)
    kiota = lax.broadcasted_iota(jnp.int32, (_T, _K), 1)
    q = jnp.min(jnp.where(dist == mind[:, None], kiota, _K), axis=1)
    q_ref[0, 0] = q
    oh = (q[None, :] == lax.broadcasted_iota(jnp.int32, (_K, _T), 0))
    ohf = oh.astype(jnp.float32)
    zq = lax.dot_general(w, ohf, (((0,), (0,)), ((), ())),
                         preferred_element_type=jnp.float32)
    zq_ref[0] = zq


def kernel(z_e, weights):
    N, D, H, W = z_e.shape
    T = H * W
    zc = z_e.reshape(N, D, T)
    q3, zq = pl.pallas_call(
        _vq_body,
        grid=(N,),
        in_specs=[
            pl.BlockSpec((1, D, T), lambda n: (n, 0, 0)),
            pl.BlockSpec((_K, D), lambda n: (0, 0)),
        ],
        out_specs=[
            pl.BlockSpec((1, 1, T), lambda n: (n, 0, 0)),
            pl.BlockSpec((1, D, T), lambda n: (n, 0, 0)),
        ],
        out_shape=[
            jax.ShapeDtypeStruct((N, 1, T), jnp.int32),
            jax.ShapeDtypeStruct((N, D, T), jnp.float32),
        ],
    )(zc, weights)
    return q3.reshape(N, H, W), zq.reshape(N, D, H, W)


# speedup vs baseline: 35.8166x; 35.8166x over previous
import jax
import jax.numpy as jnp
from jax import lax
from jax.experimental import pallas as pl
from jax.experimental.pallas import tpu as pltpu

_K = 1024
_D = 64
_T = 1024


def _vq_body(z_ref, w_ref, wt_ref, q_ref, zq_ref):
    z = z_ref[0]
    w = w_ref[...]
    wt = wt_ref[...]
    s = lax.dot_general(w, z, (((1,), (0,)), ((), ())),
                        preferred_element_type=jnp.float32)
    z2 = jnp.sum(z * z, axis=0)
    w2 = jnp.sum(w * w, axis=1)
    dist = (z2[None, :] - 2.0 * s) + w2[:, None]
    mind = jnp.min(dist, axis=0)
    kiota = lax.broadcasted_iota(jnp.int32, (_K, _T), 0)
    q = jnp.min(jnp.where(dist == mind[None, :], kiota, _K), axis=0)
    q_ref[0, 0] = q
    ohf = (kiota == q[None, :]).astype(jnp.float32)
    zq = lax.dot_general(wt, ohf, (((1,), (0,)), ((), ())),
                         preferred_element_type=jnp.float32)
    zq_ref[0] = zq


def kernel(z_e, weights):
    N, D, H, W = z_e.shape
    T = H * W
    zc = z_e.reshape(N, D, T)
    wt = weights.T
    q3, zq = pl.pallas_call(
        _vq_body,
        grid=(N,),
        in_specs=[
            pl.BlockSpec((1, D, T), lambda n: (n, 0, 0)),
            pl.BlockSpec((_K, D), lambda n: (0, 0)),
            pl.BlockSpec((D, _K), lambda n: (0, 0)),
        ],
        out_specs=[
            pl.BlockSpec((1, 1, T), lambda n: (n, 0, 0)),
            pl.BlockSpec((1, D, T), lambda n: (n, 0, 0)),
        ],
        out_shape=[
            jax.ShapeDtypeStruct((N, 1, T), jnp.int32),
            jax.ShapeDtypeStruct((N, D, T), jnp.float32),
        ],
    )(zc, weights, wt)
    return q3.reshape(N, H, W), zq.reshape(N, D, H, W)
